# Initial kernel scaffold; baseline (speedup 1.0000x reference)
import jax
import jax.numpy as jnp
from jax.experimental import pallas as pl


def kernel(src_z, dst_z, k, e1_weight):
    raise NotImplementedError("write your pallas kernel here")


# speedup vs baseline: 3.9586x; 3.9586x over previous
import functools

import jax
import jax.numpy as jnp
from jax import lax
from jax.experimental import pallas as pl
from jax.experimental.pallas import tpu as pltpu
from jax.experimental.pallas import tpu_sc as plsc

_NC, _NS, _L = 2, 16, 16
_NW = _NC * _NS
_C = 16


def _build(N, K, D):
    n_chunks = N // _C
    assert n_chunks * _C == N and K == _L
    base_t, extra = divmod(n_chunks, _NW)
    KD = K * D
    nd = D // _L

    mesh = plsc.VectorSubcoreMesh(
        core_axis_name="c", subcore_axis_name="s",
        num_cores=_NC, num_subcores=_NS)

    @functools.partial(
        pl.kernel,
        out_type=jax.ShapeDtypeStruct((N, KD), jnp.float32),
        mesh=mesh,
        scratch_types=[
            pltpu.VMEM((_C, K), jnp.int32),
            pltpu.VMEM((_C,), jnp.int32),
            pltpu.VMEM((2, D), jnp.float32),
            pltpu.VMEM((_C, KD), jnp.float32),
        ],
    )
    def run(src_hbm, dst_hbm, w_hbm, out_hbm, szv, dzv, wv, obuf):
        wid = lax.axis_index("s") * _NC + lax.axis_index("c")
        pltpu.sync_copy(w_hbm, wv)
        w0 = [wv[0, pl.ds(c * _L, _L)] for c in range(nd)]
        w1 = [wv[1, pl.ds(c * _L, _L)] for c in range(nd)]
        dw = [w1[c] - w0[c] for c in range(nd)]
        nt = base_t + jnp.where(wid < extra, 1, 0)

        def chunk_body(t, carry):
            r0 = (wid + t * _NW) * _C
            pltpu.sync_copy(src_hbm.at[pl.ds(r0, _C)], szv)
            pltpu.sync_copy(dst_hbm.at[pl.ds(r0, _C)], dzv)
            dvec = dzv[...]
            for r in range(_C):
                srow = szv[r, :]
                mask = srow == jnp.full((_L,), dvec[r], jnp.int32)
                mf = jnp.where(mask, jnp.full((_L,), 1.0, jnp.float32),
                               jnp.full((_L,), 0.0, jnp.float32))
                for j in range(K):
                    m = mf[j]
                    for c in range(nd):
                        obuf[r, pl.ds(j * D + c * _L, _L)] = (
                            w0[c] + m * dw[c])
            pltpu.sync_copy(obuf, out_hbm.at[pl.ds(r0, _C)])
            return carry

        lax.fori_loop(0, nt, chunk_body, 0)

    return run


def kernel(src_z, dst_z, k, e1_weight):
    N, K = src_z.shape
    D = e1_weight.shape[1]
    run = _build(N, K, D)
    out = run(src_z, dst_z, e1_weight)
    return out.reshape(N, K, D)


# speedup vs baseline: 4.8498x; 1.2251x over previous
import functools

import jax
import jax.numpy as jnp
from jax import lax
from jax.experimental import pallas as pl
from jax.experimental.pallas import tpu as pltpu
from jax.experimental.pallas import tpu_sc as plsc

_NC, _NS, _L = 2, 16, 16
_NW = _NC * _NS
_C = 16


def _build(N, K, D):
    n_chunks = N // _C
    assert n_chunks * _C == N and K == _L
    nt = -(-n_chunks // _NW)
    assert nt % 2 == 0 and nt >= 6
    KD = K * D
    nd = D // _L

    mesh = plsc.VectorSubcoreMesh(
        core_axis_name="c", subcore_axis_name="s",
        num_cores=_NC, num_subcores=_NS)

    @functools.partial(
        pl.kernel,
        out_type=jax.ShapeDtypeStruct((N, KD), jnp.float32),
        mesh=mesh,
        scratch_types=[
            pltpu.VMEM((_C, K), jnp.int32),
            pltpu.VMEM((_C, K), jnp.int32),
            pltpu.VMEM((_C,), jnp.int32),
            pltpu.VMEM((_C,), jnp.int32),
            pltpu.VMEM((2, D), jnp.float32),
            pltpu.VMEM((_C, KD), jnp.float32),
            pltpu.VMEM((_C, KD), jnp.float32),
            pltpu.SemaphoreType.DMA,
            pltpu.SemaphoreType.DMA,
            pltpu.SemaphoreType.DMA,
            pltpu.SemaphoreType.DMA,
        ],
    )
    def run(src_hbm, dst_hbm, w_hbm, out_hbm,
            szv0, szv1, dzv0, dzv1, wv, ob0, ob1, si0, si1, so0, so1):
        wid = lax.axis_index("s") * _NC + lax.axis_index("c")
        pltpu.sync_copy(w_hbm, wv)
        w0 = [wv[0, pl.ds(c * _L, _L)] for c in range(nd)]
        w1 = [wv[1, pl.ds(c * _L, _L)] for c in range(nd)]
        dw = [w1[c] - w0[c] for c in range(nd)]
        ones = jnp.full((_L,), 1.0, jnp.float32)
        zeros = jnp.full((_L,), 0.0, jnp.float32)
        bufs = ((szv0, dzv0, ob0, si0, so0), (szv1, dzv1, ob1, si1, so1))

        def r0_of(t):
            cid = wid + t * _NW
            return jnp.where(cid < n_chunks, cid, cid - _NW) * _C

        def issue_in(t, p):
            szv, dzv, _, si, _ = bufs[p]
            r0 = r0_of(t)
            pltpu.async_copy(src_hbm.at[pl.ds(r0, _C)], szv, si)
            pltpu.async_copy(dst_hbm.at[pl.ds(r0, _C)], dzv, si)

        def wait_in(p):
            szv, dzv, _, si, _ = bufs[p]
            pltpu.make_async_copy(src_hbm.at[pl.ds(0, _C)], szv, si).wait()
            pltpu.make_async_copy(dst_hbm.at[pl.ds(0, _C)], dzv, si).wait()

        def issue_out(t, p):
            _, _, ob, _, so = bufs[p]
            pltpu.async_copy(ob, out_hbm.at[pl.ds(r0_of(t), _C)], so)

        def wait_out(p):
            _, _, ob, _, so = bufs[p]
            pltpu.make_async_copy(ob, out_hbm.at[pl.ds(0, _C)], so).wait()

        def compute(p):
            szv, dzv, ob, _, _ = bufs[p]
            dvec = dzv[...]
            for r in range(_C):
                srow = szv[r, :]
                mask = srow == jnp.full((_L,), dvec[r], jnp.int32)
                mf = jnp.where(mask, ones, zeros)
                for j in range(K):
                    m = mf[j]
                    for c in range(nd):
                        ob[r, pl.ds(j * D + c * _L, _L)] = w0[c] + m * dw[c]

        issue_in(0, 0)
        issue_in(1, 1)
        for p in (0, 1):
            wait_in(p)
            compute(p)
            issue_out(p, p)
            issue_in(p + 2, p)

        def super_body(s, carry):
            for p in (0, 1):
                t = 2 * s + p
                wait_in(p)
                wait_out(p)
                compute(p)
                issue_out(t, p)
                issue_in(t + 2, p)
            return carry

        lax.fori_loop(1, nt // 2 - 1, super_body, 0)

        for p in (0, 1):
            wait_in(p)
            wait_out(p)
            compute(p)
            issue_out(nt - 2 + p, p)
        wait_out(0)
        wait_out(1)

    return run


def kernel(src_z, dst_z, k, e1_weight):
    N, K = src_z.shape
    D = e1_weight.shape[1]
    run = _build(N, K, D)
    out = run(src_z, dst_z, e1_weight)
    return out.reshape(N, K, D)
